# baseline (speedup 1.0000x reference)
import jax
import jax.numpy as jnp
from jax.experimental import pallas as pl
from jax.experimental.pallas import tpu as pltpu

B, T, D = 16, 2048, 1024


def _laststep_body(lens_ref, payload_ref, out_ref, sems):
    copies = []
    for b in range(B):
        row = (lens_ref[b] - 1) & (T - 1)
        copies.append(
            pltpu.make_async_copy(
                payload_ref.at[b, row], out_ref.at[b], sems.at[b]
            )
        )
    for c in copies:
        c.start()
    for c in copies:
        c.wait()


def kernel(payload, seq_lens):
    return pl.pallas_call(
        _laststep_body,
        in_specs=[
            pl.BlockSpec(memory_space=pltpu.SMEM),
            pl.BlockSpec(memory_space=pl.ANY),
        ],
        out_specs=pl.BlockSpec(memory_space=pl.ANY),
        out_shape=jax.ShapeDtypeStruct((B, D), jnp.float32),
        scratch_shapes=[pltpu.SemaphoreType.DMA((B,))],
    )(seq_lens.astype(jnp.int32), payload)
